# baseline (speedup 1.0000x reference)
import functools

import jax
import jax.numpy as jnp
from jax import lax
from jax.experimental import pallas as pl
from jax.experimental.pallas import tpu as pltpu
from jax.experimental.pallas import tpu_sc as plsc

_B, _S, _H = 1, 2048, 768
_E, _K, _I = 8, 2, 2048
_BLK = 256
_NB = 24
_ROWS = _NB * _BLK
_CH = 512
_NEG = -1e30
_TB = 256
_NTB = _S // _TB

_NP = _S * _K
_NSC = 16
_PPS = _NP // _NSC
_NW = 32
_RPW = _ROWS // _NW
_GCH = 64
_ZPS = _ROWS // _NSC


def _router_body(x_ref, wg_ref, posw_ref, gw_ref, meta_ref):
    x = x_ref[...]
    logits = jax.lax.dot_general(x, wg_ref[...], (((1,), (1,)), ((), ())),
                                 preferred_element_type=jnp.float32)
    ii = jax.lax.broadcasted_iota(jnp.int32, (_S, _E), 1)
    m0 = jnp.max(logits, axis=1, keepdims=True)
    i0 = jnp.min(jnp.where(logits == m0, ii, _E), axis=1, keepdims=True)
    lm = jnp.where(ii == i0, _NEG, logits)
    m1 = jnp.max(lm, axis=1, keepdims=True)
    i1 = jnp.min(jnp.where(lm == m1, ii, _E), axis=1, keepdims=True)
    g0 = 1.0 / (1.0 + jnp.exp(m1 - m0))
    g1 = 1.0 - g0

    oh0 = (ii == i0).astype(jnp.float32)
    oh1 = (ii == i1).astype(jnp.float32)

    lr = jax.lax.broadcasted_iota(jnp.int32, (_CH, _CH), 0)
    lc = jax.lax.broadcasted_iota(jnp.int32, (_CH, _CH), 1)
    ltri = (lc < lr).astype(jnp.float32)
    carry = jnp.zeros((1, _E), jnp.float32)
    ranks = []
    for oh in (oh0, oh1):
        for c in range(_S // _CH):
            blk = oh[c * _CH:(c + 1) * _CH, :]
            local = jax.lax.dot_general(
                ltri, blk, (((1,), (0,)), ((), ())),
                preferred_element_type=jnp.float32) + carry
            ranks.append(jnp.sum(local * blk, axis=1, keepdims=True))
            carry = carry + jnp.sum(blk, axis=0, keepdims=True)
    counts = carry

    nblk = jnp.floor((counts + (_BLK - 1)) / _BLK)
    er = jax.lax.broadcasted_iota(jnp.int32, (_E, _E), 0)
    ec = jax.lax.broadcasted_iota(jnp.int32, (_E, _E), 1)
    upper = (er < ec).astype(jnp.float32)
    off = _BLK * jax.lax.dot_general(nblk, upper, (((1,), (0,)), ((), ())),
                                     preferred_element_type=jnp.float32)

    rank0 = jnp.concatenate(ranks[:_S // _CH], axis=0)
    rank1 = jnp.concatenate(ranks[_S // _CH:], axis=0)
    pos0 = jnp.sum(oh0 * off, axis=1, keepdims=True) + rank0
    pos1 = jnp.sum(oh1 * off, axis=1, keepdims=True) + rank1
    ci = jax.lax.broadcasted_iota(jnp.int32, (_S, _E), 1)
    posw_ref[...] = jnp.where(
        ci == 0, pos0, jnp.where(ci == 1, pos1, 0.0)).astype(jnp.int32)
    gw_ref[...] = jnp.where(ci == 0, g0, jnp.where(ci == 1, g1, 0.0))

    nb_used = jnp.sum(nblk, axis=1, keepdims=True)
    bi = jax.lax.broadcasted_iota(jnp.int32, (128, _E), 0).astype(jnp.float32)
    row = jnp.minimum(bi, nb_used - 1.0) * _BLK
    offb = off * jnp.ones((128, _E), jnp.float32)
    eob = jnp.sum((row >= offb).astype(jnp.float32), axis=1,
                  keepdims=True) - 1.0
    mc = jax.lax.broadcasted_iota(jnp.int32, (128, _E), 1)
    meta_ref[...] = jnp.where(
        mc == 0, eob, jnp.where(mc == 1, nb_used, 0.0)).astype(jnp.int32)


def _router_call(x, wg):
    return pl.pallas_call(
        _router_body,
        in_specs=[
            pl.BlockSpec((_S, _H), lambda: (0, 0)),
            pl.BlockSpec((_E, _H), lambda: (0, 0)),
        ],
        out_specs=[
            pl.BlockSpec((_S, _E), lambda: (0, 0)),
            pl.BlockSpec((_S, _E), lambda: (0, 0)),
            pl.BlockSpec((128, _E), lambda: (0, 0)),
        ],
        out_shape=[
            jax.ShapeDtypeStruct((_S, _E), jnp.int32),
            jax.ShapeDtypeStruct((_S, _E), jnp.float32),
            jax.ShapeDtypeStruct((128, _E), jnp.int32),
        ],
    )(x, wg)


def _dispatch_sc_body(pos_hbm, x_hbm, xs_hbm, posv, tokv, zbuf, idxb, rowsv,
                      shared_tok, sem):
    c = lax.axis_index("c")
    s = lax.axis_index("s")
    wid = s * 2 + c

    for j in range(_ZPS // 16):
        zbuf[pl.ds(j * 16, 16)] = jnp.zeros((16,), jnp.int32)
    pltpu.sync_copy(zbuf, shared_tok.at[pl.ds(s * _ZPS, _ZPS)])
    plsc.subcore_barrier()

    pltpu.sync_copy(pos_hbm.at[pl.ds(s * (_PPS // 128), _PPS // 128)], posv)
    for j in range(_PPS // 128):
        for i in range(128 // 16):
            p16 = lax.iota(jnp.int32, 16) + (s * _PPS + j * 128 + i * 16)
            tokv[j, pl.ds(i * 16, 16)] = p16 & (_S - 1)
        pltpu.sync_copy(tokv.at[j], shared_tok.at[posv.at[j]], add=True)
    plsc.subcore_barrier()

    for g in range(_RPW // _GCH):
        row0 = wid * _RPW + g * _GCH
        pltpu.sync_copy(shared_tok.at[pl.ds(row0, _GCH)], idxb)
        pltpu.async_copy(x_hbm.at[idxb], rowsv, sem).wait()
        pltpu.sync_copy(rowsv, xs_hbm.at[pl.ds(row0, _GCH)])


def _dispatch_sc(pos2, x):
    mesh = plsc.VectorSubcoreMesh(core_axis_name="c", subcore_axis_name="s")
    kfn = functools.partial(
        pl.kernel, mesh=mesh,
        out_type=jax.ShapeDtypeStruct((_ROWS, _H), jnp.float32),
        scratch_types=[
            pltpu.VMEM((_PPS // 128, 128), jnp.int32),
            pltpu.VMEM((_PPS // 128, 128), jnp.int32),
            pltpu.VMEM((_ZPS,), jnp.int32),
            pltpu.VMEM((_GCH,), jnp.int32),
            pltpu.VMEM((_GCH, _H), jnp.float32),
            pltpu.VMEM_SHARED((_ROWS,), jnp.int32),
            pltpu.SemaphoreType.DMA,
        ],
    )(_dispatch_sc_body)
    return kfn(pos2, x)


def _ffn_body(m_ref, xs_ref, wgate_ref, wup_ref, wdown_ref, ys_ref):
    b = pl.program_id(0)
    nb = m_ref[0, 1]

    @pl.when(b < nb)
    def _compute():
        xs = xs_ref[...]
        wge = wgate_ref[0]
        wue = wup_ref[0]
        wde = wdown_ref[0]
        g = jax.lax.dot_general(xs, wge, (((1,), (1,)), ((), ())),
                                preferred_element_type=jnp.float32)
        u = jax.lax.dot_general(xs, wue, (((1,), (1,)), ((), ())),
                                preferred_element_type=jnp.float32)
        h = jax.nn.silu(g) * u
        y = jax.lax.dot_general(h, wde, (((1,), (1,)), ((), ())),
                                preferred_element_type=jnp.float32)
        ys_ref[...] = y.astype(jnp.bfloat16)

    @pl.when(b >= nb)
    def _zero():
        ys_ref[...] = jnp.zeros((_BLK, _H), jnp.bfloat16)


def _ffn_call(meta, xs, wgate, wup, wdown):
    grid_spec = pltpu.PrefetchScalarGridSpec(
        num_scalar_prefetch=1,
        grid=(_NB,),
        in_specs=[
            pl.BlockSpec((_BLK, _H), lambda b, m: (b, 0)),
            pl.BlockSpec((1, _I, _H), lambda b, m: (m[b, 0], 0, 0)),
            pl.BlockSpec((1, _I, _H), lambda b, m: (m[b, 0], 0, 0)),
            pl.BlockSpec((1, _H, _I), lambda b, m: (m[b, 0], 0, 0)),
        ],
        out_specs=pl.BlockSpec((_BLK, _H), lambda b, m: (b, 0)),
    )
    return pl.pallas_call(
        _ffn_body,
        grid_spec=grid_spec,
        out_shape=jax.ShapeDtypeStruct((_ROWS, _H), jnp.bfloat16),
        compiler_params=pltpu.CompilerParams(
            dimension_semantics=("arbitrary",)),
    )(meta, xs, wgate, wup, wdown)


def _shared_body(x_ref, wsg_ref, wsu_ref, wsd_ref, sh_ref):
    xb = x_ref[...]
    sg = jax.lax.dot_general(xb, wsg_ref[...], (((1,), (1,)), ((), ())),
                             preferred_element_type=jnp.float32)
    su = jax.lax.dot_general(xb, wsu_ref[...], (((1,), (1,)), ((), ())),
                             preferred_element_type=jnp.float32)
    sh = jax.nn.silu(sg) * su
    sh_ref[...] = jax.lax.dot_general(sh, wsd_ref[...], (((1,), (1,)), ((), ())),
                                      preferred_element_type=jnp.float32)


def _shared_call(x, wsg, wsu, wsd):
    return pl.pallas_call(
        _shared_body,
        grid=(_NTB,),
        in_specs=[
            pl.BlockSpec((_TB, _H), lambda tb: (tb, 0)),
            pl.BlockSpec((_I, _H), lambda tb: (0, 0)),
            pl.BlockSpec((_I, _H), lambda tb: (0, 0)),
            pl.BlockSpec((_H, _I), lambda tb: (0, 0)),
        ],
        out_specs=pl.BlockSpec((_TB, _H), lambda tb: (tb, 0)),
        out_shape=jax.ShapeDtypeStruct((_S, _H), jnp.float32),
        compiler_params=pltpu.CompilerParams(
            dimension_semantics=("arbitrary",)),
    )(x, wsg, wsu, wsd)


def _combine_body(sh_ref, posw_ref, gw_ref, ys_ref, out_ref):
    p0 = posw_ref[:, 0:1]
    p1 = posw_ref[:, 1:2]
    g0 = gw_ref[:, 0:1]
    g1 = gw_ref[:, 1:2]
    jj = jax.lax.broadcasted_iota(jnp.int32, (_TB, _ROWS), 1)
    gmat = (jnp.where(jj == p0, g0, 0.0)
            + jnp.where(jj == p1, g1, 0.0)).astype(jnp.bfloat16)
    yc = jax.lax.dot_general(gmat, ys_ref[...], (((1,), (0,)), ((), ())),
                             preferred_element_type=jnp.float32)
    out_ref[...] = sh_ref[...] + yc


def _combine_call(sh, posw, gw, ys):
    return pl.pallas_call(
        _combine_body,
        grid=(_NTB,),
        in_specs=[
            pl.BlockSpec((_TB, _H), lambda tb: (tb, 0)),
            pl.BlockSpec((_TB, _E), lambda tb: (tb, 0)),
            pl.BlockSpec((_TB, _E), lambda tb: (tb, 0)),
            pl.BlockSpec((_ROWS, _H), lambda tb: (0, 0)),
        ],
        out_specs=pl.BlockSpec((_TB, _H), lambda tb: (tb, 0)),
        out_shape=jax.ShapeDtypeStruct((_S, _H), jnp.float32),
        compiler_params=pltpu.CompilerParams(
            dimension_semantics=("arbitrary",)),
    )(sh, posw, gw, ys)


@jax.jit
def kernel(hidden_states, Wg, W_gate, W_up, W_down, Ws_gate, Ws_up, Ws_down):
    b, s, h = hidden_states.shape
    x = hidden_states.reshape(s, h)
    posw, gw, meta = _router_call(x, Wg)
    pos2 = jnp.concatenate([posw[:, 0], posw[:, 1]]).reshape(_NP // 128, 128)
    xs = _dispatch_sc(pos2, x)
    sh = _shared_call(x, Ws_gate, Ws_up, Ws_down)
    ys = _ffn_call(meta, xs, W_gate, W_up, W_down)
    out = _combine_call(sh, posw, gw, ys)
    return out.reshape(b, s, h)


# speedup vs baseline: 1.3951x; 1.3951x over previous
import jax
import jax.numpy as jnp
from jax.experimental import pallas as pl
from jax.experimental.pallas import tpu as pltpu

_B, _S, _H = 1, 2048, 768
_E, _K, _I = 8, 2, 2048
_BLK = 256
_NB = 24
_ROWS = _NB * _BLK
_CH = 512
_NEG = -1e30
_TB = 256
_NTB = _S // _TB


def _router_body(x_ref, wg_ref, posw_ref, gw_ref, meta_ref):
    x = x_ref[...]
    logits = jax.lax.dot_general(x, wg_ref[...], (((1,), (1,)), ((), ())),
                                 preferred_element_type=jnp.float32)
    ii = jax.lax.broadcasted_iota(jnp.int32, (_S, _E), 1)
    m0 = jnp.max(logits, axis=1, keepdims=True)
    i0 = jnp.min(jnp.where(logits == m0, ii, _E), axis=1, keepdims=True)
    lm = jnp.where(ii == i0, _NEG, logits)
    m1 = jnp.max(lm, axis=1, keepdims=True)
    i1 = jnp.min(jnp.where(lm == m1, ii, _E), axis=1, keepdims=True)
    g0 = 1.0 / (1.0 + jnp.exp(m1 - m0))
    g1 = 1.0 - g0

    oh0 = (ii == i0).astype(jnp.float32)
    oh1 = (ii == i1).astype(jnp.float32)

    lr = jax.lax.broadcasted_iota(jnp.int32, (_CH, _CH), 0)
    lc = jax.lax.broadcasted_iota(jnp.int32, (_CH, _CH), 1)
    ltri = (lc < lr).astype(jnp.float32)
    carry = jnp.zeros((1, _E), jnp.float32)
    ranks = []
    for oh in (oh0, oh1):
        for c in range(_S // _CH):
            blk = oh[c * _CH:(c + 1) * _CH, :]
            local = jax.lax.dot_general(
                ltri, blk, (((1,), (0,)), ((), ())),
                preferred_element_type=jnp.float32) + carry
            ranks.append(jnp.sum(local * blk, axis=1, keepdims=True))
            carry = carry + jnp.sum(blk, axis=0, keepdims=True)
    counts = carry

    nblk = jnp.floor((counts + (_BLK - 1)) / _BLK)
    er = jax.lax.broadcasted_iota(jnp.int32, (_E, _E), 0)
    ec = jax.lax.broadcasted_iota(jnp.int32, (_E, _E), 1)
    upper = (er < ec).astype(jnp.float32)
    off = _BLK * jax.lax.dot_general(nblk, upper, (((1,), (0,)), ((), ())),
                                     preferred_element_type=jnp.float32)

    rank0 = jnp.concatenate(ranks[:_S // _CH], axis=0)
    rank1 = jnp.concatenate(ranks[_S // _CH:], axis=0)
    pos0 = jnp.sum(oh0 * off, axis=1, keepdims=True) + rank0
    pos1 = jnp.sum(oh1 * off, axis=1, keepdims=True) + rank1
    ci = jax.lax.broadcasted_iota(jnp.int32, (_S, _E), 1)
    posw_ref[...] = jnp.where(
        ci == 0, pos0, jnp.where(ci == 1, pos1, 0.0)).astype(jnp.int32)
    gw_ref[...] = jnp.where(ci == 0, g0, jnp.where(ci == 1, g1, 0.0))

    nb_used = jnp.sum(nblk, axis=1, keepdims=True)
    bi = jax.lax.broadcasted_iota(jnp.int32, (128, _E), 0).astype(jnp.float32)
    row = jnp.minimum(bi, nb_used - 1.0) * _BLK
    offb = off * jnp.ones((128, _E), jnp.float32)
    eob = jnp.sum((row >= offb).astype(jnp.float32), axis=1,
                  keepdims=True) - 1.0
    mc = jax.lax.broadcasted_iota(jnp.int32, (128, _E), 1)
    meta_ref[...] = jnp.where(
        mc == 0, eob, jnp.where(mc == 1, nb_used, 0.0)).astype(jnp.int32)


def _router_call(x, wg):
    return pl.pallas_call(
        _router_body,
        in_specs=[
            pl.BlockSpec((_S, _H), lambda: (0, 0)),
            pl.BlockSpec((_E, _H), lambda: (0, 0)),
        ],
        out_specs=[
            pl.BlockSpec((_S, _E), lambda: (0, 0)),
            pl.BlockSpec((_S, _E), lambda: (0, 0)),
            pl.BlockSpec((128, _E), lambda: (0, 0)),
        ],
        out_shape=[
            jax.ShapeDtypeStruct((_S, _E), jnp.int32),
            jax.ShapeDtypeStruct((_S, _E), jnp.float32),
            jax.ShapeDtypeStruct((128, _E), jnp.int32),
        ],
    )(x, wg)


def _ffn_body(m_ref, posw_ref, x_ref, wgate_ref, wup_ref, wdown_ref, ys_ref):
    b = pl.program_id(0)
    nb = m_ref[0, 1]

    @pl.when(b < nb)
    def _compute():
        p0 = posw_ref[:, 0:1]
        p1 = posw_ref[:, 1:2]
        rr = jax.lax.broadcasted_iota(jnp.int32, (_S, _BLK), 1) + b * _BLK
        m2 = ((rr == p0) | (rr == p1)).astype(jnp.float32)
        xs = jax.lax.dot_general(m2, x_ref[...], (((0,), (0,)), ((), ())),
                                 preferred_element_type=jnp.float32)
        wge = wgate_ref[0]
        wue = wup_ref[0]
        wde = wdown_ref[0]
        g = jax.lax.dot_general(xs, wge, (((1,), (1,)), ((), ())),
                                preferred_element_type=jnp.float32)
        u = jax.lax.dot_general(xs, wue, (((1,), (1,)), ((), ())),
                                preferred_element_type=jnp.float32)
        h = jax.nn.silu(g) * u
        y = jax.lax.dot_general(h, wde, (((1,), (1,)), ((), ())),
                                preferred_element_type=jnp.float32)
        ys_ref[...] = y.astype(jnp.bfloat16)

    @pl.when(b >= nb)
    def _zero():
        ys_ref[...] = jnp.zeros((_BLK, _H), jnp.bfloat16)


def _ffn_call(meta, posw, x, wgate, wup, wdown):
    grid_spec = pltpu.PrefetchScalarGridSpec(
        num_scalar_prefetch=1,
        grid=(_NB,),
        in_specs=[
            pl.BlockSpec((_S, _E), lambda b, m: (0, 0)),
            pl.BlockSpec((_S, _H), lambda b, m: (0, 0)),
            pl.BlockSpec((1, _I, _H), lambda b, m: (m[b, 0], 0, 0)),
            pl.BlockSpec((1, _I, _H), lambda b, m: (m[b, 0], 0, 0)),
            pl.BlockSpec((1, _H, _I), lambda b, m: (m[b, 0], 0, 0)),
        ],
        out_specs=pl.BlockSpec((_BLK, _H), lambda b, m: (b, 0)),
    )
    return pl.pallas_call(
        _ffn_body,
        grid_spec=grid_spec,
        out_shape=jax.ShapeDtypeStruct((_ROWS, _H), jnp.bfloat16),
        compiler_params=pltpu.CompilerParams(
            dimension_semantics=("arbitrary",)),
    )(meta, posw, x, wgate, wup, wdown)


_CC = 1024
_NCC = _ROWS // _CC


def _combine_body(m_ref, x_ref, wsg_ref, wsu_ref, wsd_ref, posw_ref, gw_ref,
                  ys_ref, out_ref):
    nb = m_ref[0, 1]
    xb = x_ref[...]
    sg = jax.lax.dot_general(xb, wsg_ref[...], (((1,), (1,)), ((), ())),
                             preferred_element_type=jnp.float32)
    su = jax.lax.dot_general(xb, wsu_ref[...], (((1,), (1,)), ((), ())),
                             preferred_element_type=jnp.float32)
    sh = jax.nn.silu(sg) * su
    out_ref[...] = jax.lax.dot_general(sh, wsd_ref[...], (((1,), (1,)), ((), ())),
                                       preferred_element_type=jnp.float32)

    p0 = posw_ref[:, 0:1]
    p1 = posw_ref[:, 1:2]
    g0 = gw_ref[:, 0:1]
    g1 = gw_ref[:, 1:2]
    jj = jax.lax.broadcasted_iota(jnp.int32, (_TB, _CC), 1)
    for c in range(_NCC):
        @pl.when(c * (_CC // _BLK) < nb)
        def _chunk(c=c):
            jc = jj + c * _CC
            gmat = (jnp.where(jc == p0, g0, 0.0)
                    + jnp.where(jc == p1, g1, 0.0)).astype(jnp.bfloat16)
            yc = jax.lax.dot_general(
                gmat, ys_ref[c * _CC:(c + 1) * _CC, :],
                (((1,), (0,)), ((), ())),
                preferred_element_type=jnp.float32)
            out_ref[...] += yc


def _combine_call(meta, x, wsg, wsu, wsd, posw, gw, ys):
    grid_spec = pltpu.PrefetchScalarGridSpec(
        num_scalar_prefetch=1,
        grid=(_NTB,),
        in_specs=[
            pl.BlockSpec((_TB, _H), lambda tb, m: (tb, 0)),
            pl.BlockSpec((_I, _H), lambda tb, m: (0, 0)),
            pl.BlockSpec((_I, _H), lambda tb, m: (0, 0)),
            pl.BlockSpec((_H, _I), lambda tb, m: (0, 0)),
            pl.BlockSpec((_TB, _E), lambda tb, m: (tb, 0)),
            pl.BlockSpec((_TB, _E), lambda tb, m: (tb, 0)),
            pl.BlockSpec((_ROWS, _H), lambda tb, m: (0, 0)),
        ],
        out_specs=pl.BlockSpec((_TB, _H), lambda tb, m: (tb, 0)),
    )
    return pl.pallas_call(
        _combine_body,
        grid_spec=grid_spec,
        out_shape=jax.ShapeDtypeStruct((_S, _H), jnp.float32),
        compiler_params=pltpu.CompilerParams(
            dimension_semantics=("arbitrary",)),
    )(meta, x, wsg, wsu, wsd, posw, gw, ys)


@jax.jit
def kernel(hidden_states, Wg, W_gate, W_up, W_down, Ws_gate, Ws_up, Ws_down):
    b, s, h = hidden_states.shape
    x = hidden_states.reshape(s, h)
    posw, gw, meta = _router_call(x, Wg)
    ys = _ffn_call(meta, posw, x, W_gate, W_up, W_down)
    out = _combine_call(meta, x, Ws_gate, Ws_up, Ws_down, posw, gw, ys)
    return out.reshape(b, s, h)


# speedup vs baseline: 1.6342x; 1.1714x over previous
import jax
import jax.numpy as jnp
from jax.experimental import pallas as pl
from jax.experimental.pallas import tpu as pltpu

_B, _S, _H = 1, 2048, 768
_E, _K, _I = 8, 2, 2048
_BLK = 256
_NB = 24
_ROWS = _NB * _BLK
_CH = 512
_NEG = -1e30
_TB = 256
_NTB = _S // _TB


def _router_body(x_ref, wg_ref, posw_ref, gw_ref, meta_ref):
    x = x_ref[...]
    logits = jax.lax.dot_general(x, wg_ref[...], (((1,), (1,)), ((), ())),
                                 preferred_element_type=jnp.float32)
    ii = jax.lax.broadcasted_iota(jnp.int32, (_S, _E), 1)
    m0 = jnp.max(logits, axis=1, keepdims=True)
    i0 = jnp.min(jnp.where(logits == m0, ii, _E), axis=1, keepdims=True)
    lm = jnp.where(ii == i0, _NEG, logits)
    m1 = jnp.max(lm, axis=1, keepdims=True)
    i1 = jnp.min(jnp.where(lm == m1, ii, _E), axis=1, keepdims=True)
    g0 = 1.0 / (1.0 + jnp.exp(m1 - m0))
    g1 = 1.0 - g0

    oh0 = (ii == i0).astype(jnp.float32)
    oh1 = (ii == i1).astype(jnp.float32)

    lr = jax.lax.broadcasted_iota(jnp.int32, (_CH, _CH), 0)
    lc = jax.lax.broadcasted_iota(jnp.int32, (_CH, _CH), 1)
    ltri = (lc < lr).astype(jnp.float32)
    carry = jnp.zeros((1, _E), jnp.float32)
    ranks = []
    for oh in (oh0, oh1):
        for c in range(_S // _CH):
            blk = oh[c * _CH:(c + 1) * _CH, :]
            local = jax.lax.dot_general(
                ltri, blk, (((1,), (0,)), ((), ())),
                preferred_element_type=jnp.float32) + carry
            ranks.append(jnp.sum(local * blk, axis=1, keepdims=True))
            carry = carry + jnp.sum(blk, axis=0, keepdims=True)
    counts = carry

    nblk = jnp.floor((counts + (_BLK - 1)) / _BLK)
    er = jax.lax.broadcasted_iota(jnp.int32, (_E, _E), 0)
    ec = jax.lax.broadcasted_iota(jnp.int32, (_E, _E), 1)
    upper = (er < ec).astype(jnp.float32)
    off = _BLK * jax.lax.dot_general(nblk, upper, (((1,), (0,)), ((), ())),
                                     preferred_element_type=jnp.float32)

    rank0 = jnp.concatenate(ranks[:_S // _CH], axis=0)
    rank1 = jnp.concatenate(ranks[_S // _CH:], axis=0)
    pos0 = jnp.sum(oh0 * off, axis=1, keepdims=True) + rank0
    pos1 = jnp.sum(oh1 * off, axis=1, keepdims=True) + rank1
    ci = jax.lax.broadcasted_iota(jnp.int32, (_S, _E), 1)
    posw_ref[...] = jnp.where(
        ci == 0, pos0, jnp.where(ci == 1, pos1, 0.0)).astype(jnp.int32)
    gw_ref[...] = jnp.where(ci == 0, g0, jnp.where(ci == 1, g1, 0.0))

    nb_used = jnp.sum(nblk, axis=1, keepdims=True)
    bi = jax.lax.broadcasted_iota(jnp.int32, (128, _E), 0).astype(jnp.float32)
    row = jnp.minimum(bi, nb_used - 1.0) * _BLK
    offb = off * jnp.ones((128, _E), jnp.float32)
    eob = jnp.sum((row >= offb).astype(jnp.float32), axis=1,
                  keepdims=True) - 1.0
    mc = jax.lax.broadcasted_iota(jnp.int32, (128, _E), 1)
    meta_ref[...] = jnp.where(
        mc == 0, eob, jnp.where(mc == 1, nb_used, 0.0)).astype(jnp.int32)


def _router_call(x, wg):
    return pl.pallas_call(
        _router_body,
        in_specs=[
            pl.BlockSpec((_S, _H), lambda: (0, 0)),
            pl.BlockSpec((_E, _H), lambda: (0, 0)),
        ],
        out_specs=[
            pl.BlockSpec((_S, _E), lambda: (0, 0)),
            pl.BlockSpec((_S, _E), lambda: (0, 0)),
            pl.BlockSpec((128, _E), lambda: (0, 0)),
        ],
        out_shape=[
            jax.ShapeDtypeStruct((_S, _E), jnp.int32),
            jax.ShapeDtypeStruct((_S, _E), jnp.float32),
            jax.ShapeDtypeStruct((128, _E), jnp.int32),
        ],
    )(x, wg)


def _ffn_body(m_ref, posw_ref, gw_ref, x_ref, wgate_ref, wup_ref, wdown_ref,
              rout_ref):
    b = pl.program_id(0)
    nb = m_ref[0, 1]

    @pl.when(b == 0)
    def _init():
        rout_ref[...] = jnp.zeros((_S, _H), jnp.float32)

    @pl.when(b < nb)
    def _compute():
        p0 = posw_ref[:, 0:1]
        p1 = posw_ref[:, 1:2]
        rr = jax.lax.broadcasted_iota(jnp.int32, (_S, _BLK), 1) + b * _BLK
        eq0 = rr == p0
        eq1 = rr == p1
        m2 = (eq0 | eq1).astype(jnp.float32)
        xs = jax.lax.dot_general(m2, x_ref[...], (((0,), (0,)), ((), ())),
                                 preferred_element_type=jnp.float32)
        wge = wgate_ref[0]
        wue = wup_ref[0]
        wde = wdown_ref[0]
        g = jax.lax.dot_general(xs, wge, (((1,), (1,)), ((), ())),
                                preferred_element_type=jnp.float32)
        u = jax.lax.dot_general(xs, wue, (((1,), (1,)), ((), ())),
                                preferred_element_type=jnp.float32)
        h = jax.nn.silu(g) * u
        y = jax.lax.dot_general(h, wde, (((1,), (1,)), ((), ())),
                                preferred_element_type=jnp.float32)
        m2g = (jnp.where(eq0, gw_ref[:, 0:1], 0.0)
               + jnp.where(eq1, gw_ref[:, 1:2], 0.0))
        rout_ref[...] += jax.lax.dot_general(
            m2g, y, (((1,), (0,)), ((), ())),
            preferred_element_type=jnp.float32)


def _ffn_call(meta, posw, gw, x, wgate, wup, wdown):
    grid_spec = pltpu.PrefetchScalarGridSpec(
        num_scalar_prefetch=1,
        grid=(_NB,),
        in_specs=[
            pl.BlockSpec((_S, _E), lambda b, m: (0, 0)),
            pl.BlockSpec((_S, _E), lambda b, m: (0, 0)),
            pl.BlockSpec((_S, _H), lambda b, m: (0, 0)),
            pl.BlockSpec((1, _I, _H), lambda b, m: (m[b, 0], 0, 0)),
            pl.BlockSpec((1, _I, _H), lambda b, m: (m[b, 0], 0, 0)),
            pl.BlockSpec((1, _H, _I), lambda b, m: (m[b, 0], 0, 0)),
        ],
        out_specs=pl.BlockSpec((_S, _H), lambda b, m: (0, 0)),
    )
    return pl.pallas_call(
        _ffn_body,
        grid_spec=grid_spec,
        out_shape=jax.ShapeDtypeStruct((_S, _H), jnp.float32),
        compiler_params=pltpu.CompilerParams(
            dimension_semantics=("arbitrary",)),
    )(meta, posw, gw, x, wgate, wup, wdown)


def _shared_body(x_ref, wsg_ref, wsu_ref, wsd_ref, rout_ref, out_ref):
    xb = x_ref[...]
    sg = jax.lax.dot_general(xb, wsg_ref[...], (((1,), (1,)), ((), ())),
                             preferred_element_type=jnp.float32)
    su = jax.lax.dot_general(xb, wsu_ref[...], (((1,), (1,)), ((), ())),
                             preferred_element_type=jnp.float32)
    sh = jax.nn.silu(sg) * su
    shared = jax.lax.dot_general(sh, wsd_ref[...], (((1,), (1,)), ((), ())),
                                 preferred_element_type=jnp.float32)
    out_ref[...] = shared + rout_ref[...]


def _shared_call(x, wsg, wsu, wsd, rout):
    return pl.pallas_call(
        _shared_body,
        grid=(_NTB,),
        in_specs=[
            pl.BlockSpec((_TB, _H), lambda tb: (tb, 0)),
            pl.BlockSpec((_I, _H), lambda tb: (0, 0)),
            pl.BlockSpec((_I, _H), lambda tb: (0, 0)),
            pl.BlockSpec((_H, _I), lambda tb: (0, 0)),
            pl.BlockSpec((_TB, _H), lambda tb: (tb, 0)),
        ],
        out_specs=pl.BlockSpec((_TB, _H), lambda tb: (tb, 0)),
        out_shape=jax.ShapeDtypeStruct((_S, _H), jnp.float32),
        compiler_params=pltpu.CompilerParams(
            dimension_semantics=("arbitrary",)),
    )(x, wsg, wsu, wsd, rout)


@jax.jit
def kernel(hidden_states, Wg, W_gate, W_up, W_down, Ws_gate, Ws_up, Ws_down):
    b, s, h = hidden_states.shape
    x = hidden_states.reshape(s, h)
    posw, gw, meta = _router_call(x, Wg)
    rout = _ffn_call(meta, posw, gw, x, W_gate, W_up, W_down)
    out = _shared_call(x, Ws_gate, Ws_up, Ws_down, rout)
    return out.reshape(b, s, h)


# speedup vs baseline: 1.7017x; 1.0413x over previous
import jax
import jax.numpy as jnp
from jax.experimental import pallas as pl
from jax.experimental.pallas import tpu as pltpu

_B, _S, _H = 1, 2048, 768
_E, _K, _I = 8, 2, 2048
_BLK = 256
_NB = 24
_ROWS = _NB * _BLK
_CH = 512
_NEG = -1e30
_TB = 256
_NTB = _S // _TB


def _router_body(x_ref, wg_ref, posw_ref, gw_ref, meta_ref):
    x = x_ref[...]
    logits = jax.lax.dot_general(x, wg_ref[...], (((1,), (1,)), ((), ())),
                                 preferred_element_type=jnp.float32)
    ii = jax.lax.broadcasted_iota(jnp.int32, (_S, _E), 1)
    m0 = jnp.max(logits, axis=1, keepdims=True)
    i0 = jnp.min(jnp.where(logits == m0, ii, _E), axis=1, keepdims=True)
    lm = jnp.where(ii == i0, _NEG, logits)
    m1 = jnp.max(lm, axis=1, keepdims=True)
    i1 = jnp.min(jnp.where(lm == m1, ii, _E), axis=1, keepdims=True)
    g0 = 1.0 / (1.0 + jnp.exp(m1 - m0))
    g1 = 1.0 - g0

    oh0 = (ii == i0).astype(jnp.float32)
    oh1 = (ii == i1).astype(jnp.float32)

    lr = jax.lax.broadcasted_iota(jnp.int32, (_CH, _CH), 0)
    lc = jax.lax.broadcasted_iota(jnp.int32, (_CH, _CH), 1)
    ltri = (lc < lr).astype(jnp.float32)
    carry = jnp.zeros((1, _E), jnp.float32)
    ranks = []
    for oh in (oh0, oh1):
        for c in range(_S // _CH):
            blk = oh[c * _CH:(c + 1) * _CH, :]
            local = jax.lax.dot_general(
                ltri, blk, (((1,), (0,)), ((), ())),
                preferred_element_type=jnp.float32) + carry
            ranks.append(jnp.sum(local * blk, axis=1, keepdims=True))
            carry = carry + jnp.sum(blk, axis=0, keepdims=True)
    counts = carry

    nblk = jnp.floor((counts + (_BLK - 1)) / _BLK)
    er = jax.lax.broadcasted_iota(jnp.int32, (_E, _E), 0)
    ec = jax.lax.broadcasted_iota(jnp.int32, (_E, _E), 1)
    upper = (er < ec).astype(jnp.float32)
    off = _BLK * jax.lax.dot_general(nblk, upper, (((1,), (0,)), ((), ())),
                                     preferred_element_type=jnp.float32)

    rank0 = jnp.concatenate(ranks[:_S // _CH], axis=0)
    rank1 = jnp.concatenate(ranks[_S // _CH:], axis=0)
    pos0 = jnp.sum(oh0 * off, axis=1, keepdims=True) + rank0
    pos1 = jnp.sum(oh1 * off, axis=1, keepdims=True) + rank1
    ci = jax.lax.broadcasted_iota(jnp.int32, (_S, _E), 1)
    posw_ref[...] = jnp.where(
        ci == 0, pos0, jnp.where(ci == 1, pos1, 0.0)).astype(jnp.int32)
    gw_ref[...] = jnp.where(ci == 0, g0, jnp.where(ci == 1, g1, 0.0))

    nb_used = jnp.sum(nblk, axis=1, keepdims=True)
    bi = jax.lax.broadcasted_iota(jnp.int32, (128, _E), 0).astype(jnp.float32)
    ec = jax.lax.broadcasted_iota(jnp.int32, (128, _E), 1).astype(jnp.float32)
    row = jnp.minimum(bi, nb_used - 1.0) * _BLK
    offb = off * jnp.ones((128, _E), jnp.float32)
    eob = jnp.sum((row >= offb).astype(jnp.float32), axis=1,
                  keepdims=True) - 1.0
    inb = (bi[:, 0:1] < nb_used).astype(jnp.float32)
    first = jnp.minimum(
        jnp.sum((bi * _BLK == offb).astype(jnp.float32), axis=1,
                keepdims=True), 1.0) * inb
    br = jax.lax.broadcasted_iota(jnp.int32, (128, 128), 0)
    bc = jax.lax.broadcasted_iota(jnp.int32, (128, 128), 1)
    itri = (bc <= br).astype(jnp.float32)
    seq = jax.lax.dot_general(itri, first, (((1,), (0,)), ((), ())),
                              preferred_element_type=jnp.float32) - 1.0
    par = seq - 2.0 * jnp.floor(seq * 0.5)
    nblkb = nblk * jnp.ones((128, _E), jnp.float32)
    eobb = eob * jnp.ones((128, _E), jnp.float32)
    nxt = jnp.min(jnp.where((ec > eobb) & (nblkb > 0.0), ec, float(_E)),
                  axis=1, keepdims=True)
    hasn = (nxt < float(_E)).astype(jnp.float32)
    nxt = jnp.minimum(nxt, float(_E - 1))
    mc = jax.lax.broadcasted_iota(jnp.int32, (128, _E), 1)
    meta = jnp.where(mc == 0, eob, 0.0)
    meta = jnp.where(mc == 1, nb_used, meta)
    meta = jnp.where(mc == 2, first, meta)
    meta = jnp.where(mc == 3, par, meta)
    meta = jnp.where(mc == 4, nxt, meta)
    meta = jnp.where(mc == 5, hasn, meta)
    meta_ref[...] = meta.astype(jnp.int32)


def _router_call(x, wg):
    return pl.pallas_call(
        _router_body,
        in_specs=[
            pl.BlockSpec((_S, _H), lambda: (0, 0)),
            pl.BlockSpec((_E, _H), lambda: (0, 0)),
        ],
        out_specs=[
            pl.BlockSpec((_S, _E), lambda: (0, 0)),
            pl.BlockSpec((_S, _E), lambda: (0, 0)),
            pl.BlockSpec((128, _E), lambda: (0, 0)),
        ],
        out_shape=[
            jax.ShapeDtypeStruct((_S, _E), jnp.int32),
            jax.ShapeDtypeStruct((_S, _E), jnp.float32),
            jax.ShapeDtypeStruct((128, _E), jnp.int32),
        ],
    )(x, wg)


def _issue_weights(wgate_ref, wup_ref, wdown_ref, wg_s, wu_s, wd_s, sem,
                   e, slot):
    pltpu.make_async_copy(wgate_ref.at[e], wg_s.at[slot], sem.at[slot]).start()
    pltpu.make_async_copy(wup_ref.at[e], wu_s.at[slot], sem.at[slot]).start()
    pltpu.make_async_copy(wdown_ref.at[e], wd_s.at[slot], sem.at[slot]).start()


def _wait_weights(wgate_ref, wup_ref, wdown_ref, wg_s, wu_s, wd_s, sem,
                  e, slot):
    pltpu.make_async_copy(wgate_ref.at[e], wg_s.at[slot], sem.at[slot]).wait()
    pltpu.make_async_copy(wup_ref.at[e], wu_s.at[slot], sem.at[slot]).wait()
    pltpu.make_async_copy(wdown_ref.at[e], wd_s.at[slot], sem.at[slot]).wait()


def _ffn_body(m_ref, posw_ref, gw_ref, x_ref, wgate_ref, wup_ref, wdown_ref,
              rout_ref, wg_s, wu_s, wd_s, sem):
    b = pl.program_id(0)
    nb = m_ref[0, 1]
    e = m_ref[b, 0]
    first = m_ref[b, 2]
    par = m_ref[b, 3]
    nxt = m_ref[b, 4]
    hasn = m_ref[b, 5]

    @pl.when(b == 0)
    def _init():
        rout_ref[...] = jnp.zeros((_S, _H), jnp.float32)
        _issue_weights(wgate_ref, wup_ref, wdown_ref, wg_s, wu_s, wd_s, sem,
                       e, 0)

    @pl.when((first == 1) & (hasn == 1))
    def _prefetch_next():
        _issue_weights(wgate_ref, wup_ref, wdown_ref, wg_s, wu_s, wd_s, sem,
                       nxt, 1 - par)

    @pl.when(first == 1)
    def _wait_cur():
        _wait_weights(wgate_ref, wup_ref, wdown_ref, wg_s, wu_s, wd_s, sem,
                      e, par)

    @pl.when(b < nb)
    def _compute():
        p0 = posw_ref[:, 0:1]
        p1 = posw_ref[:, 1:2]
        rr = jax.lax.broadcasted_iota(jnp.int32, (_S, _BLK), 1) + b * _BLK
        eq0 = rr == p0
        eq1 = rr == p1
        m2 = (eq0 | eq1).astype(jnp.float32)
        xs = jax.lax.dot_general(m2, x_ref[...], (((0,), (0,)), ((), ())),
                                 preferred_element_type=jnp.float32)
        wge = wg_s[par]
        wue = wu_s[par]
        wde = wd_s[par]
        g = jax.lax.dot_general(xs, wge, (((1,), (1,)), ((), ())),
                                preferred_element_type=jnp.float32)
        u = jax.lax.dot_general(xs, wue, (((1,), (1,)), ((), ())),
                                preferred_element_type=jnp.float32)
        h = jax.nn.silu(g) * u
        y = jax.lax.dot_general(h, wde, (((1,), (1,)), ((), ())),
                                preferred_element_type=jnp.float32)
        m2g = (jnp.where(eq0, gw_ref[:, 0:1], 0.0)
               + jnp.where(eq1, gw_ref[:, 1:2], 0.0))
        rout_ref[...] += jax.lax.dot_general(
            m2g, y, (((1,), (0,)), ((), ())),
            preferred_element_type=jnp.float32)


def _ffn_call(meta, posw, gw, x, wgate, wup, wdown):
    grid_spec = pltpu.PrefetchScalarGridSpec(
        num_scalar_prefetch=1,
        grid=(_NB,),
        in_specs=[
            pl.BlockSpec((_S, _E), lambda b, m: (0, 0)),
            pl.BlockSpec((_S, _E), lambda b, m: (0, 0)),
            pl.BlockSpec((_S, _H), lambda b, m: (0, 0)),
            pl.BlockSpec(memory_space=pl.ANY),
            pl.BlockSpec(memory_space=pl.ANY),
            pl.BlockSpec(memory_space=pl.ANY),
        ],
        out_specs=pl.BlockSpec((_S, _H), lambda b, m: (0, 0)),
        scratch_shapes=[
            pltpu.VMEM((2, _I, _H), jnp.float32),
            pltpu.VMEM((2, _I, _H), jnp.float32),
            pltpu.VMEM((2, _H, _I), jnp.float32),
            pltpu.SemaphoreType.DMA((2,)),
        ],
    )
    return pl.pallas_call(
        _ffn_body,
        grid_spec=grid_spec,
        out_shape=jax.ShapeDtypeStruct((_S, _H), jnp.float32),
        compiler_params=pltpu.CompilerParams(
            dimension_semantics=("arbitrary",)),
    )(meta, posw, gw, x, wgate, wup, wdown)


def _shared_body(x_ref, wsg_ref, wsu_ref, wsd_ref, rout_ref, out_ref):
    xb = x_ref[...]
    sg = jax.lax.dot_general(xb, wsg_ref[...], (((1,), (1,)), ((), ())),
                             preferred_element_type=jnp.float32)
    su = jax.lax.dot_general(xb, wsu_ref[...], (((1,), (1,)), ((), ())),
                             preferred_element_type=jnp.float32)
    sh = jax.nn.silu(sg) * su
    shared = jax.lax.dot_general(sh, wsd_ref[...], (((1,), (1,)), ((), ())),
                                 preferred_element_type=jnp.float32)
    out_ref[...] = shared + rout_ref[...]


def _shared_call(x, wsg, wsu, wsd, rout):
    return pl.pallas_call(
        _shared_body,
        grid=(_NTB,),
        in_specs=[
            pl.BlockSpec((_TB, _H), lambda tb: (tb, 0)),
            pl.BlockSpec((_I, _H), lambda tb: (0, 0)),
            pl.BlockSpec((_I, _H), lambda tb: (0, 0)),
            pl.BlockSpec((_H, _I), lambda tb: (0, 0)),
            pl.BlockSpec((_TB, _H), lambda tb: (tb, 0)),
        ],
        out_specs=pl.BlockSpec((_TB, _H), lambda tb: (tb, 0)),
        out_shape=jax.ShapeDtypeStruct((_S, _H), jnp.float32),
        compiler_params=pltpu.CompilerParams(
            dimension_semantics=("arbitrary",)),
    )(x, wsg, wsu, wsd, rout)


@jax.jit
def kernel(hidden_states, Wg, W_gate, W_up, W_down, Ws_gate, Ws_up, Ws_down):
    b, s, h = hidden_states.shape
    x = hidden_states.reshape(s, h)
    posw, gw, meta = _router_call(x, Wg)
    rout = _ffn_call(meta, posw, gw, x, W_gate, W_up, W_down)
    out = _shared_call(x, Ws_gate, Ws_up, Ws_down, rout)
    return out.reshape(b, s, h)


# speedup vs baseline: 1.7018x; 1.0000x over previous
import jax
import jax.numpy as jnp
from jax.experimental import pallas as pl
from jax.experimental.pallas import tpu as pltpu

_B, _S, _H = 1, 2048, 768
_E, _K, _I = 8, 2, 2048
_BLK = 256
_NB = 24
_ROWS = _NB * _BLK
_CH = 512
_NEG = -1e30
_TB = 256
_NTB = _S // _TB


def _router_body(x_ref, wg_ref, posw_ref, gw_ref, meta_ref):
    x = x_ref[...]
    logits = jax.lax.dot_general(x, wg_ref[...], (((1,), (1,)), ((), ())),
                                 preferred_element_type=jnp.float32)
    ii = jax.lax.broadcasted_iota(jnp.int32, (_S, _E), 1)
    m0 = jnp.max(logits, axis=1, keepdims=True)
    i0 = jnp.min(jnp.where(logits == m0, ii, _E), axis=1, keepdims=True)
    lm = jnp.where(ii == i0, _NEG, logits)
    m1 = jnp.max(lm, axis=1, keepdims=True)
    i1 = jnp.min(jnp.where(lm == m1, ii, _E), axis=1, keepdims=True)
    g0 = 1.0 / (1.0 + jnp.exp(m1 - m0))
    g1 = 1.0 - g0

    oh0 = (ii == i0).astype(jnp.float32)
    oh1 = (ii == i1).astype(jnp.float32)

    lr = jax.lax.broadcasted_iota(jnp.int32, (_CH, _CH), 0)
    lc = jax.lax.broadcasted_iota(jnp.int32, (_CH, _CH), 1)
    ltri = (lc < lr).astype(jnp.float32)
    chunks = []
    for oh in (oh0, oh1):
        for c in range(_S // _CH):
            chunks.append(oh[c * _CH:(c + 1) * _CH, :])
    sums = [jnp.sum(blk, axis=0, keepdims=True) for blk in chunks]
    carries = [jnp.zeros((1, _E), jnp.float32)]
    for sm in sums[:-1]:
        carries.append(carries[-1] + sm)
    counts = carries[-1] + sums[-1]
    ranks = []
    for blk, carry in zip(chunks, carries):
        local = jax.lax.dot_general(
            ltri, blk, (((1,), (0,)), ((), ())),
            preferred_element_type=jnp.float32) + carry
        ranks.append(jnp.sum(local * blk, axis=1, keepdims=True))

    nblk = jnp.floor((counts + (_BLK - 1)) / _BLK)
    er = jax.lax.broadcasted_iota(jnp.int32, (_E, _E), 0)
    ec = jax.lax.broadcasted_iota(jnp.int32, (_E, _E), 1)
    upper = (er < ec).astype(jnp.float32)
    off = _BLK * jax.lax.dot_general(nblk, upper, (((1,), (0,)), ((), ())),
                                     preferred_element_type=jnp.float32)

    rank0 = jnp.concatenate(ranks[:_S // _CH], axis=0)
    rank1 = jnp.concatenate(ranks[_S // _CH:], axis=0)
    pos0 = jnp.sum(oh0 * off, axis=1, keepdims=True) + rank0
    pos1 = jnp.sum(oh1 * off, axis=1, keepdims=True) + rank1
    ci = jax.lax.broadcasted_iota(jnp.int32, (_S, _E), 1)
    posw_ref[...] = jnp.where(
        ci == 0, pos0, jnp.where(ci == 1, pos1, 0.0)).astype(jnp.int32)
    gw_ref[...] = jnp.where(ci == 0, g0, jnp.where(ci == 1, g1, 0.0))

    nb_used = jnp.sum(nblk, axis=1, keepdims=True)
    bi = jax.lax.broadcasted_iota(jnp.int32, (128, _E), 0).astype(jnp.float32)
    ec = jax.lax.broadcasted_iota(jnp.int32, (128, _E), 1).astype(jnp.float32)
    row = jnp.minimum(bi, nb_used - 1.0) * _BLK
    offb = off * jnp.ones((128, _E), jnp.float32)
    eob = jnp.sum((row >= offb).astype(jnp.float32), axis=1,
                  keepdims=True) - 1.0
    inb = (bi[:, 0:1] < nb_used).astype(jnp.float32)
    first = jnp.minimum(
        jnp.sum((bi * _BLK == offb).astype(jnp.float32), axis=1,
                keepdims=True), 1.0) * inb
    br = jax.lax.broadcasted_iota(jnp.int32, (128, 128), 0)
    bc = jax.lax.broadcasted_iota(jnp.int32, (128, 128), 1)
    itri = (bc <= br).astype(jnp.float32)
    seq = jax.lax.dot_general(itri, first, (((1,), (0,)), ((), ())),
                              preferred_element_type=jnp.float32) - 1.0
    par = seq - 2.0 * jnp.floor(seq * 0.5)
    nblkb = nblk * jnp.ones((128, _E), jnp.float32)
    eobb = eob * jnp.ones((128, _E), jnp.float32)
    nxt = jnp.min(jnp.where((ec > eobb) & (nblkb > 0.0), ec, float(_E)),
                  axis=1, keepdims=True)
    hasn = (nxt < float(_E)).astype(jnp.float32)
    nxt = jnp.minimum(nxt, float(_E - 1))
    mc = jax.lax.broadcasted_iota(jnp.int32, (128, _E), 1)
    meta = jnp.where(mc == 0, eob, 0.0)
    meta = jnp.where(mc == 1, nb_used, meta)
    meta = jnp.where(mc == 2, first, meta)
    meta = jnp.where(mc == 3, par, meta)
    meta = jnp.where(mc == 4, nxt, meta)
    meta = jnp.where(mc == 5, hasn, meta)
    meta_ref[...] = meta.astype(jnp.int32)


def _router_call(x, wg):
    return pl.pallas_call(
        _router_body,
        in_specs=[
            pl.BlockSpec((_S, _H), lambda: (0, 0)),
            pl.BlockSpec((_E, _H), lambda: (0, 0)),
        ],
        out_specs=[
            pl.BlockSpec((_S, _E), lambda: (0, 0)),
            pl.BlockSpec((_S, _E), lambda: (0, 0)),
            pl.BlockSpec((128, _E), lambda: (0, 0)),
        ],
        out_shape=[
            jax.ShapeDtypeStruct((_S, _E), jnp.int32),
            jax.ShapeDtypeStruct((_S, _E), jnp.float32),
            jax.ShapeDtypeStruct((128, _E), jnp.int32),
        ],
    )(x, wg)


def _issue_weights(wgate_ref, wup_ref, wdown_ref, wg_s, wu_s, wd_s, sem,
                   e, slot):
    pltpu.make_async_copy(wgate_ref.at[e], wg_s.at[slot], sem.at[slot]).start()
    pltpu.make_async_copy(wup_ref.at[e], wu_s.at[slot], sem.at[slot]).start()
    pltpu.make_async_copy(wdown_ref.at[e], wd_s.at[slot], sem.at[slot]).start()


def _wait_weights(wgate_ref, wup_ref, wdown_ref, wg_s, wu_s, wd_s, sem,
                  e, slot):
    pltpu.make_async_copy(wgate_ref.at[e], wg_s.at[slot], sem.at[slot]).wait()
    pltpu.make_async_copy(wup_ref.at[e], wu_s.at[slot], sem.at[slot]).wait()
    pltpu.make_async_copy(wdown_ref.at[e], wd_s.at[slot], sem.at[slot]).wait()


def _ffn_body(m_ref, posw_ref, gw_ref, x_ref, wgate_ref, wup_ref, wdown_ref,
              rout_ref, wg_s, wu_s, wd_s, sem):
    b = pl.program_id(0)
    nb = m_ref[0, 1]
    e = m_ref[b, 0]
    first = m_ref[b, 2]
    par = m_ref[b, 3]
    nxt = m_ref[b, 4]
    hasn = m_ref[b, 5]

    @pl.when(b == 0)
    def _init():
        rout_ref[...] = jnp.zeros((_S, _H), jnp.float32)
        _issue_weights(wgate_ref, wup_ref, wdown_ref, wg_s, wu_s, wd_s, sem,
                       e, 0)

    @pl.when((first == 1) & (hasn == 1))
    def _prefetch_next():
        _issue_weights(wgate_ref, wup_ref, wdown_ref, wg_s, wu_s, wd_s, sem,
                       nxt, 1 - par)

    @pl.when(first == 1)
    def _wait_cur():
        _wait_weights(wgate_ref, wup_ref, wdown_ref, wg_s, wu_s, wd_s, sem,
                      e, par)

    @pl.when(b < nb)
    def _compute():
        p0 = posw_ref[:, 0:1]
        p1 = posw_ref[:, 1:2]
        rr = jax.lax.broadcasted_iota(jnp.int32, (_S, _BLK), 1) + b * _BLK
        eq0 = rr == p0
        eq1 = rr == p1
        m2 = (eq0 | eq1).astype(jnp.float32)
        xs = jax.lax.dot_general(m2, x_ref[...], (((0,), (0,)), ((), ())),
                                 preferred_element_type=jnp.float32)
        wge = wg_s[par]
        wue = wu_s[par]
        wde = wd_s[par]
        g = jax.lax.dot_general(xs, wge, (((1,), (1,)), ((), ())),
                                preferred_element_type=jnp.float32)
        u = jax.lax.dot_general(xs, wue, (((1,), (1,)), ((), ())),
                                preferred_element_type=jnp.float32)
        h = jax.nn.silu(g) * u
        y = jax.lax.dot_general(h, wde, (((1,), (1,)), ((), ())),
                                preferred_element_type=jnp.float32)
        m2g = (jnp.where(eq0, gw_ref[:, 0:1], 0.0)
               + jnp.where(eq1, gw_ref[:, 1:2], 0.0))
        rout_ref[...] += jax.lax.dot_general(
            m2g, y, (((1,), (0,)), ((), ())),
            preferred_element_type=jnp.float32)


def _ffn_call(meta, posw, gw, x, wgate, wup, wdown):
    grid_spec = pltpu.PrefetchScalarGridSpec(
        num_scalar_prefetch=1,
        grid=(_NB,),
        in_specs=[
            pl.BlockSpec((_S, _E), lambda b, m: (0, 0)),
            pl.BlockSpec((_S, _E), lambda b, m: (0, 0)),
            pl.BlockSpec((_S, _H), lambda b, m: (0, 0)),
            pl.BlockSpec(memory_space=pl.ANY),
            pl.BlockSpec(memory_space=pl.ANY),
            pl.BlockSpec(memory_space=pl.ANY),
        ],
        out_specs=pl.BlockSpec((_S, _H), lambda b, m: (0, 0)),
        scratch_shapes=[
            pltpu.VMEM((2, _I, _H), jnp.float32),
            pltpu.VMEM((2, _I, _H), jnp.float32),
            pltpu.VMEM((2, _H, _I), jnp.float32),
            pltpu.SemaphoreType.DMA((2,)),
        ],
    )
    return pl.pallas_call(
        _ffn_body,
        grid_spec=grid_spec,
        out_shape=jax.ShapeDtypeStruct((_S, _H), jnp.float32),
        compiler_params=pltpu.CompilerParams(
            dimension_semantics=("arbitrary",)),
    )(meta, posw, gw, x, wgate, wup, wdown)


def _shared_body(x_ref, wsg_ref, wsu_ref, wsd_ref, rout_ref, out_ref):
    xb = x_ref[...]
    sg = jax.lax.dot_general(xb, wsg_ref[...], (((1,), (1,)), ((), ())),
                             preferred_element_type=jnp.float32)
    su = jax.lax.dot_general(xb, wsu_ref[...], (((1,), (1,)), ((), ())),
                             preferred_element_type=jnp.float32)
    sh = jax.nn.silu(sg) * su
    shared = jax.lax.dot_general(sh, wsd_ref[...], (((1,), (1,)), ((), ())),
                                 preferred_element_type=jnp.float32)
    out_ref[...] = shared + rout_ref[...]


def _shared_call(x, wsg, wsu, wsd, rout):
    return pl.pallas_call(
        _shared_body,
        grid=(_NTB,),
        in_specs=[
            pl.BlockSpec((_TB, _H), lambda tb: (tb, 0)),
            pl.BlockSpec((_I, _H), lambda tb: (0, 0)),
            pl.BlockSpec((_I, _H), lambda tb: (0, 0)),
            pl.BlockSpec((_H, _I), lambda tb: (0, 0)),
            pl.BlockSpec((_TB, _H), lambda tb: (tb, 0)),
        ],
        out_specs=pl.BlockSpec((_TB, _H), lambda tb: (tb, 0)),
        out_shape=jax.ShapeDtypeStruct((_S, _H), jnp.float32),
        compiler_params=pltpu.CompilerParams(
            dimension_semantics=("arbitrary",)),
    )(x, wsg, wsu, wsd, rout)


@jax.jit
def kernel(hidden_states, Wg, W_gate, W_up, W_down, Ws_gate, Ws_up, Ws_down):
    b, s, h = hidden_states.shape
    x = hidden_states.reshape(s, h)
    posw, gw, meta = _router_call(x, Wg)
    rout = _ffn_call(meta, posw, gw, x, W_gate, W_up, W_down)
    out = _shared_call(x, Ws_gate, Ws_up, Ws_down, rout)
    return out.reshape(b, s, h)
